# baseline (speedup 1.0000x reference)
import jax
import jax.numpy as jnp
from jax import lax
from jax.experimental import pallas as pl
from jax.experimental.pallas import tpu as pltpu
from jax.experimental.pallas import tpu_sc as plsc

T = 2048
HID = 1024
FF = 512
E = 8
K = 2
SFF = 1024
BLK = 256
NP = T * K
G = (NP + E * (BLK - 1) + BLK - 1) // BLK
RS = G * BLK

NW = 32
PW = NP // NW
CH = 64


def _pack2(xbf):
    half = xbf.shape[-1] // 2
    lo = pltpu.bitcast(xbf[:, :half], jnp.int16).astype(jnp.int32) & 0xFFFF
    hi = pltpu.bitcast(xbf[:, half:], jnp.int16).astype(jnp.int32) & 0xFFFF
    return lo | (hi << 16)


def _unpack2(xi):
    lo = pltpu.bitcast((xi & 0xFFFF).astype(jnp.int16), jnp.bfloat16)
    hi = pltpu.bitcast(((xi >> 16) & 0xFFFF).astype(jnp.int16), jnp.bfloat16)
    return jnp.concatenate([lo, hi], axis=-1)


def _router_body(x_ref, gw_ref, b_ref, p1_ref, p2_ref, w1_ref, w2_ref, be_ref,
                 xb_ref):
    x = x_ref[...]
    xb_ref[...] = _pack2(x.astype(jnp.bfloat16))
    logits = jnp.dot(x, gw_ref[...], preferred_element_type=jnp.float32)
    m = jnp.max(logits, axis=-1, keepdims=True)
    ex = jnp.exp(logits - m)
    scores = ex / jnp.sum(ex, axis=-1, keepdims=True)
    choice = scores + b_ref[...]
    lane = lax.broadcasted_iota(jnp.int32, (T, E), 1)
    v1 = jnp.max(choice, axis=-1, keepdims=True)
    e1 = jnp.min(jnp.where(choice == v1, lane, E), axis=-1, keepdims=True)
    ch2 = jnp.where(lane == e1, -jnp.inf, choice)
    v2 = jnp.max(ch2, axis=-1, keepdims=True)
    e2 = jnp.min(jnp.where(ch2 == v2, lane, E), axis=-1, keepdims=True)
    w1 = jnp.sum(jnp.where(lane == e1, scores, 0.0), axis=-1, keepdims=True)
    w2 = jnp.sum(jnp.where(lane == e2, scores, 0.0), axis=-1, keepdims=True)
    tot = w1 + w2
    w1_ref[...] = w1 / tot
    w2_ref[...] = w2 / tot
    ohc = jnp.concatenate(
        [(lane == e1).astype(jnp.bfloat16), (lane == e2).astype(jnp.bfloat16)],
        axis=-1)
    CK = 128
    NCK = T // CK
    rc = lax.broadcasted_iota(jnp.int32, (CK, CK), 0)
    cc = lax.broadcasted_iota(jnp.int32, (CK, CK), 1)
    tric = (cc < rc).astype(jnp.bfloat16)
    locs, tots = [], []
    for ci in range(NCK):
        blk = ohc[ci * CK:(ci + 1) * CK, :]
        loc = jnp.dot(tric, blk, preferred_element_type=jnp.float32)
        locs.append(loc)
        tots.append(loc[CK - 1:CK, :] + blk[CK - 1:CK, :].astype(jnp.float32))
    tot = jnp.concatenate(tots, axis=0)
    r16 = lax.broadcasted_iota(jnp.int32, (NCK, NCK), 0)
    c16 = lax.broadcasted_iota(jnp.int32, (NCK, NCK), 1)
    tri16 = (c16 < r16).astype(jnp.float32)
    off = jnp.dot(tri16, tot, preferred_element_type=jnp.float32)
    rank = jnp.concatenate(
        [locs[ci] + off[ci:ci + 1, :] for ci in range(NCK)], axis=0)
    c1 = rank[:, :E]
    c2 = rank[:, E:]
    allcnt = off[NCK - 1:NCK, :] + tot[NCK - 1:NCK, :]
    cnt1 = allcnt[:, :E]
    counts = cnt1 + allcnt[:, E:]
    pc = jnp.ceil(counts * (1.0 / BLK)) * BLK
    r8 = lax.broadcasted_iota(jnp.int32, (E, E), 0)
    c8 = lax.broadcasted_iota(jnp.int32, (E, E), 1)
    tri8 = (r8 < c8).astype(jnp.float32)
    seg = jnp.dot(pc, tri8, preferred_element_type=jnp.float32)
    p1 = jnp.sum(jnp.where(lane == e1, seg + c1, 0.0), axis=-1, keepdims=True)
    p2 = jnp.sum(jnp.where(lane == e2, seg + cnt1 + c2, 0.0), axis=-1,
                 keepdims=True)
    p1_ref[...] = p1.astype(jnp.int32)
    p2_ref[...] = p2.astype(jnp.int32)
    gb = lax.broadcasted_iota(jnp.int32, (G + 1, E), 0) * BLK
    segi = seg.astype(jnp.int32)
    be = jnp.sum((gb >= segi).astype(jnp.int32), axis=-1, keepdims=True) - 1
    nb = (segi[0, E - 1] + pc.astype(jnp.int32)[0, E - 1]) // BLK
    row = lax.broadcasted_iota(jnp.int32, (G + 1, 1), 0)
    be_ref[...] = jnp.where(row == G, nb, be)


def _router(x, gate_w, bias2d):
    return pl.pallas_call(
        _router_body,
        out_shape=(
            jax.ShapeDtypeStruct((T, 1), jnp.int32),
            jax.ShapeDtypeStruct((T, 1), jnp.int32),
            jax.ShapeDtypeStruct((T, 1), jnp.float32),
            jax.ShapeDtypeStruct((T, 1), jnp.float32),
            jax.ShapeDtypeStruct((G + 1, 1), jnp.int32),
            jax.ShapeDtypeStruct((T, HID // 2), jnp.int32),
        ),
    )(x, gate_w, bias2d)


def _sc_scatter(x, idx):
    mesh = plsc.VectorSubcoreMesh(core_axis_name="c", subcore_axis_name="s")

    @pl.kernel(
        out_type=jax.ShapeDtypeStruct((RS, HID // 2), jnp.int32),
        mesh=mesh,
        scratch_types=[
            pltpu.VMEM((CH,), jnp.int32),
            pltpu.VMEM((CH, HID // 2), jnp.int32),
            pltpu.SemaphoreType.DMA,
        ],
    )
    def _disp(x_hbm, idx_hbm, xs_hbm, idx_v, rows_v, sem):
        cid = lax.axis_index("c")
        sid = lax.axis_index("s")
        wid = sid * 2 + cid
        base = wid * PW
        for chk in range(PW // CH):
            off = base + chk * CH
            pltpu.sync_copy(idx_hbm.at[pl.ds(off, CH)], idx_v)
            src = lax.rem(off, T)
            pltpu.sync_copy(x_hbm.at[pl.ds(src, CH)], rows_v)
            pltpu.async_copy(rows_v, xs_hbm.at[idx_v], sem).wait()

    return _disp(x, idx)


def _sc_gather(ys, idx):
    mesh = plsc.VectorSubcoreMesh(core_axis_name="c", subcore_axis_name="s")

    @pl.kernel(
        out_type=jax.ShapeDtypeStruct((NP, HID // 2), jnp.int32),
        mesh=mesh,
        scratch_types=[
            pltpu.VMEM((CH,), jnp.int32),
            pltpu.VMEM((CH, HID // 2), jnp.int32),
            pltpu.SemaphoreType.DMA,
        ],
    )
    def _gath(ys_hbm, idx_hbm, g_hbm, idx_v, rows_v, sem):
        cid = lax.axis_index("c")
        sid = lax.axis_index("s")
        wid = sid * 2 + cid
        base = wid * PW
        for chk in range(PW // CH):
            off = base + chk * CH
            pltpu.sync_copy(idx_hbm.at[pl.ds(off, CH)], idx_v)
            pltpu.async_copy(ys_hbm.at[idx_v], rows_v, sem).wait()
            pltpu.sync_copy(rows_v, g_hbm.at[pl.ds(off, CH)])

    return _gath(ys, idx)


def _group_body(be_ref, xs_ref, wg_ref, wu_ref, wd_ref, dep_ref, ys_ref,
                wgb_ref, wub_ref, wdb_ref):
    del dep_ref
    i = pl.program_id(0)
    nb = be_ref[G]

    @pl.when(i < nb)
    def _compute():
        e = be_ref[i]
        prev_e = be_ref[jnp.maximum(i - 1, 0)]

        @pl.when((i == 0) | (e != prev_e))
        def _cache_expert():
            wgb_ref[...] = wg_ref[e].astype(jnp.bfloat16)
            wub_ref[...] = wu_ref[e].astype(jnp.bfloat16)
            wdb_ref[...] = wd_ref[e].astype(jnp.bfloat16)

        x = _unpack2(xs_ref[...])
        g = jnp.dot(x, wgb_ref[...], preferred_element_type=jnp.float32)
        u = jnp.dot(x, wub_ref[...], preferred_element_type=jnp.float32)
        h = (g * lax.logistic(g) * u).astype(jnp.bfloat16)
        y = jnp.dot(h, wdb_ref[...], preferred_element_type=jnp.float32)
        ys_ref[...] = _pack2(y.astype(jnp.bfloat16))


def _grouped(be, xs, w_gate, w_up, w_down, dep):
    grid_spec = pltpu.PrefetchScalarGridSpec(
        num_scalar_prefetch=1,
        grid=(G,),
        in_specs=[
            pl.BlockSpec((BLK, HID // 2), lambda i, be: (i, 0)),
            pl.BlockSpec((E, HID, FF), lambda i, be: (0, 0, 0)),
            pl.BlockSpec((E, HID, FF), lambda i, be: (0, 0, 0)),
            pl.BlockSpec((E, FF, HID), lambda i, be: (0, 0, 0)),
            pl.BlockSpec((8, 128), lambda i, be: (0, 0)),
        ],
        out_specs=pl.BlockSpec((BLK, HID // 2), lambda i, be: (i, 0)),
        scratch_shapes=[
            pltpu.VMEM((HID, FF), jnp.bfloat16),
            pltpu.VMEM((HID, FF), jnp.bfloat16),
            pltpu.VMEM((FF, HID), jnp.bfloat16),
        ],
    )
    return pl.pallas_call(
        _group_body,
        grid_spec=grid_spec,
        out_shape=jax.ShapeDtypeStruct((RS, HID // 2), jnp.int32),
        compiler_params=pltpu.CompilerParams(
            dimension_semantics=("arbitrary",)),
    )(be, xs, w_gate, w_up, w_down, dep)


def _shared_body(x_ref, wg_ref, wu_ref, wd_ref, d_ref, o_ref):
    del d_ref
    x = x_ref[...].astype(jnp.bfloat16)
    g = jnp.dot(x, wg_ref[...].astype(jnp.bfloat16),
                preferred_element_type=jnp.float32)
    u = jnp.dot(x, wu_ref[...].astype(jnp.bfloat16),
                preferred_element_type=jnp.float32)
    h = (g * lax.logistic(g) * u).astype(jnp.bfloat16)
    o_ref[...] = jnp.dot(h, wd_ref[...].astype(jnp.bfloat16),
                         preferred_element_type=jnp.float32)


def _shared_part(x, sw_gate, sw_up, sw_down, dep, dep_spec, off, nb):
    rb = 256
    return pl.pallas_call(
        _shared_body,
        grid=(nb,),
        in_specs=[
            pl.BlockSpec((rb, HID), lambda i: (i + off, 0)),
            pl.BlockSpec((HID, SFF), lambda i: (0, 0)),
            pl.BlockSpec((HID, SFF), lambda i: (0, 0)),
            pl.BlockSpec((SFF, HID), lambda i: (0, 0)),
            dep_spec,
        ],
        out_specs=pl.BlockSpec((rb, HID), lambda i: (i, 0)),
        out_shape=jax.ShapeDtypeStruct((nb * rb, HID), jnp.float32),
        compiler_params=pltpu.CompilerParams(
            dimension_semantics=("arbitrary",)),
    )(x, sw_gate, sw_up, sw_down, dep)


def _combine_body(sa_ref, sb_ref, g0_ref, g1_ref, w1_ref, w2_ref, o_ref):
    g0 = _unpack2(g0_ref[...]).astype(jnp.float32)
    g1 = _unpack2(g1_ref[...]).astype(jnp.float32)
    s = jnp.where(pl.program_id(0) < 4, sa_ref[...], sb_ref[...])
    o_ref[...] = s + w1_ref[...] * g0 + w2_ref[...] * g1


def _combine(sh_a, sh_b, garr, w1, w2):
    nb = 8
    rb = T // nb
    return pl.pallas_call(
        _combine_body,
        grid=(nb,),
        in_specs=[
            pl.BlockSpec((rb, HID), lambda i: (jnp.minimum(i, 3), 0)),
            pl.BlockSpec((rb, HID), lambda i: (jnp.maximum(i - 4, 0), 0)),
            pl.BlockSpec((rb, HID // 2), lambda i: (i, 0)),
            pl.BlockSpec((rb, HID // 2), lambda i: (i + nb, 0)),
            pl.BlockSpec((rb, 1), lambda i: (i, 0)),
            pl.BlockSpec((rb, 1), lambda i: (i, 0)),
        ],
        out_specs=pl.BlockSpec((rb, HID), lambda i: (i, 0)),
        out_shape=jax.ShapeDtypeStruct((T, HID), jnp.float32),
        compiler_params=pltpu.CompilerParams(
            dimension_semantics=("arbitrary",)),
    )(sh_a, sh_b, garr, garr, w1, w2)


def kernel(hidden_states, visual_token_mask, gate_w, e_score_bias,
           w_gate, w_up, w_down, sw_gate, sw_up, sw_down):
    del visual_token_mask
    x = hidden_states.reshape(T, HID)
    p1, p2, w1, w2, be, xb = _router(x, gate_w, e_score_bias.reshape(1, E))
    idx = jnp.concatenate([p1[:, 0], p2[:, 0]])
    xs = _sc_scatter(xb, idx)
    sh_a = _shared_part(x, sw_gate, sw_up, sw_down, p1,
                        pl.BlockSpec((T, 1), lambda i: (0, 0)), 0, 4)
    ys = _grouped(be[:, 0], xs, w_gate, w_up, w_down, sh_a)
    garr = _sc_gather(ys, idx)
    sh_b = _shared_part(x, sw_gate, sw_up, sw_down, ys,
                        pl.BlockSpec((8, 128), lambda i: (0, 0)), 4, 4)
    out = _combine(sh_a, sh_b, garr, w1, w2)
    return out.reshape(hidden_states.shape)


# speedup vs baseline: 1.0233x; 1.0233x over previous
import jax
import jax.numpy as jnp
from jax import lax
from jax.experimental import pallas as pl
from jax.experimental.pallas import tpu as pltpu
from jax.experimental.pallas import tpu_sc as plsc

T = 2048
HID = 1024
FF = 512
E = 8
K = 2
SFF = 1024
BLK = 256
NP = T * K
G = (NP + E * (BLK - 1) + BLK - 1) // BLK
RS = G * BLK

NW = 32
PW = NP // NW
CH = 64


def _pack2(xbf):
    half = xbf.shape[-1] // 2
    lo = pltpu.bitcast(xbf[:, :half], jnp.int16).astype(jnp.int32) & 0xFFFF
    hi = pltpu.bitcast(xbf[:, half:], jnp.int16).astype(jnp.int32) & 0xFFFF
    return lo | (hi << 16)


def _unpack2(xi):
    lo = pltpu.bitcast((xi & 0xFFFF).astype(jnp.int16), jnp.bfloat16)
    hi = pltpu.bitcast(((xi >> 16) & 0xFFFF).astype(jnp.int16), jnp.bfloat16)
    return jnp.concatenate([lo, hi], axis=-1)


def _router_body(x_ref, gw_ref, b_ref, p1_ref, p2_ref, w1_ref, w2_ref, be_ref,
                 xb_ref):
    x = x_ref[...]
    xb_ref[...] = _pack2(x.astype(jnp.bfloat16))
    logits = jnp.dot(x, gw_ref[...], preferred_element_type=jnp.float32)
    m = jnp.max(logits, axis=-1, keepdims=True)
    ex = jnp.exp(logits - m)
    scores = ex / jnp.sum(ex, axis=-1, keepdims=True)
    choice = scores + b_ref[...]
    lane = lax.broadcasted_iota(jnp.int32, (T, E), 1)
    v1 = jnp.max(choice, axis=-1, keepdims=True)
    e1 = jnp.min(jnp.where(choice == v1, lane, E), axis=-1, keepdims=True)
    ch2 = jnp.where(lane == e1, -jnp.inf, choice)
    v2 = jnp.max(ch2, axis=-1, keepdims=True)
    e2 = jnp.min(jnp.where(ch2 == v2, lane, E), axis=-1, keepdims=True)
    w1 = jnp.sum(jnp.where(lane == e1, scores, 0.0), axis=-1, keepdims=True)
    w2 = jnp.sum(jnp.where(lane == e2, scores, 0.0), axis=-1, keepdims=True)
    tot = w1 + w2
    w1_ref[...] = w1 / tot
    w2_ref[...] = w2 / tot
    ohc = jnp.concatenate(
        [(lane == e1).astype(jnp.bfloat16), (lane == e2).astype(jnp.bfloat16)],
        axis=-1)
    CK = 128
    NCK = T // CK
    rc = lax.broadcasted_iota(jnp.int32, (CK, CK), 0)
    cc = lax.broadcasted_iota(jnp.int32, (CK, CK), 1)
    tric = (cc < rc).astype(jnp.bfloat16)
    locs, tots = [], []
    for ci in range(NCK):
        blk = ohc[ci * CK:(ci + 1) * CK, :]
        loc = jnp.dot(tric, blk, preferred_element_type=jnp.float32)
        locs.append(loc)
        tots.append(loc[CK - 1:CK, :] + blk[CK - 1:CK, :].astype(jnp.float32))
    tot = jnp.concatenate(tots, axis=0)
    r16 = lax.broadcasted_iota(jnp.int32, (NCK, NCK), 0)
    c16 = lax.broadcasted_iota(jnp.int32, (NCK, NCK), 1)
    tri16 = (c16 < r16).astype(jnp.float32)
    off = jnp.dot(tri16, tot, preferred_element_type=jnp.float32)
    rank = jnp.concatenate(
        [locs[ci] + off[ci:ci + 1, :] for ci in range(NCK)], axis=0)
    c1 = rank[:, :E]
    c2 = rank[:, E:]
    allcnt = off[NCK - 1:NCK, :] + tot[NCK - 1:NCK, :]
    cnt1 = allcnt[:, :E]
    counts = cnt1 + allcnt[:, E:]
    pc = jnp.ceil(counts * (1.0 / BLK)) * BLK
    r8 = lax.broadcasted_iota(jnp.int32, (E, E), 0)
    c8 = lax.broadcasted_iota(jnp.int32, (E, E), 1)
    tri8 = (r8 < c8).astype(jnp.float32)
    seg = jnp.dot(pc, tri8, preferred_element_type=jnp.float32)
    p1 = jnp.sum(jnp.where(lane == e1, seg + c1, 0.0), axis=-1, keepdims=True)
    p2 = jnp.sum(jnp.where(lane == e2, seg + cnt1 + c2, 0.0), axis=-1,
                 keepdims=True)
    p1_ref[...] = p1.astype(jnp.int32)
    p2_ref[...] = p2.astype(jnp.int32)
    gb = lax.broadcasted_iota(jnp.int32, (G + 1, E), 0) * BLK
    segi = seg.astype(jnp.int32)
    be = jnp.sum((gb >= segi).astype(jnp.int32), axis=-1, keepdims=True) - 1
    nb = (segi[0, E - 1] + pc.astype(jnp.int32)[0, E - 1]) // BLK
    row = lax.broadcasted_iota(jnp.int32, (G + 1, 1), 0)
    be_ref[...] = jnp.where(row == G, nb, be)


def _router(x, gate_w, bias2d):
    return pl.pallas_call(
        _router_body,
        out_shape=(
            jax.ShapeDtypeStruct((T, 1), jnp.int32),
            jax.ShapeDtypeStruct((T, 1), jnp.int32),
            jax.ShapeDtypeStruct((T, 1), jnp.float32),
            jax.ShapeDtypeStruct((T, 1), jnp.float32),
            jax.ShapeDtypeStruct((G + 1, 1), jnp.int32),
            jax.ShapeDtypeStruct((T, HID // 2), jnp.int32),
        ),
    )(x, gate_w, bias2d)


def _sc_scatter(x, idx):
    mesh = plsc.VectorSubcoreMesh(core_axis_name="c", subcore_axis_name="s")

    @pl.kernel(
        out_type=jax.ShapeDtypeStruct((RS, HID // 2), jnp.int32),
        mesh=mesh,
        scratch_types=[
            pltpu.VMEM((CH,), jnp.int32),
            pltpu.VMEM((CH, HID // 2), jnp.int32),
            pltpu.SemaphoreType.DMA,
        ],
    )
    def _disp(x_hbm, idx_hbm, xs_hbm, idx_v, rows_v, sem):
        cid = lax.axis_index("c")
        sid = lax.axis_index("s")
        wid = sid * 2 + cid
        base = wid * PW
        for chk in range(PW // CH):
            off = base + chk * CH
            pltpu.sync_copy(idx_hbm.at[pl.ds(off, CH)], idx_v)
            src = lax.rem(off, T)
            pltpu.sync_copy(x_hbm.at[pl.ds(src, CH)], rows_v)
            pltpu.async_copy(rows_v, xs_hbm.at[idx_v], sem).wait()

    return _disp(x, idx)


def _sc_gather(ys, idx):
    mesh = plsc.VectorSubcoreMesh(core_axis_name="c", subcore_axis_name="s")

    @pl.kernel(
        out_type=jax.ShapeDtypeStruct((NP, HID // 2), jnp.int32),
        mesh=mesh,
        scratch_types=[
            pltpu.VMEM((CH,), jnp.int32),
            pltpu.VMEM((CH, HID // 2), jnp.int32),
            pltpu.SemaphoreType.DMA,
        ],
    )
    def _gath(ys_hbm, idx_hbm, g_hbm, idx_v, rows_v, sem):
        cid = lax.axis_index("c")
        sid = lax.axis_index("s")
        wid = sid * 2 + cid
        base = wid * PW
        for chk in range(PW // CH):
            off = base + chk * CH
            pltpu.sync_copy(idx_hbm.at[pl.ds(off, CH)], idx_v)
            pltpu.async_copy(ys_hbm.at[idx_v], rows_v, sem).wait()
            pltpu.sync_copy(rows_v, g_hbm.at[pl.ds(off, CH)])

    return _gath(ys, idx)


def _group_body(be_ref, xs_ref, wg_ref, wu_ref, wd_ref, ys_ref,
                wgb_ref, wub_ref, wdb_ref):
    i = pl.program_id(0)
    nb = be_ref[G]

    @pl.when(i < nb)
    def _compute():
        e = be_ref[i]
        prev_e = be_ref[jnp.maximum(i - 1, 0)]

        @pl.when((i == 0) | (e != prev_e))
        def _cache_expert():
            wgb_ref[...] = wg_ref[e].astype(jnp.bfloat16)
            wub_ref[...] = wu_ref[e].astype(jnp.bfloat16)
            wdb_ref[...] = wd_ref[e].astype(jnp.bfloat16)

        x = _unpack2(xs_ref[...])
        g = jnp.dot(x, wgb_ref[...], preferred_element_type=jnp.float32)
        u = jnp.dot(x, wub_ref[...], preferred_element_type=jnp.float32)
        h = (g * lax.logistic(g) * u).astype(jnp.bfloat16)
        y = jnp.dot(h, wdb_ref[...], preferred_element_type=jnp.float32)
        ys_ref[...] = _pack2(y.astype(jnp.bfloat16))


def _grouped(be, xs, w_gate, w_up, w_down):
    grid_spec = pltpu.PrefetchScalarGridSpec(
        num_scalar_prefetch=1,
        grid=(G,),
        in_specs=[
            pl.BlockSpec((BLK, HID // 2), lambda i, be: (i, 0)),
            pl.BlockSpec((E, HID, FF), lambda i, be: (0, 0, 0)),
            pl.BlockSpec((E, HID, FF), lambda i, be: (0, 0, 0)),
            pl.BlockSpec((E, FF, HID), lambda i, be: (0, 0, 0)),
        ],
        out_specs=pl.BlockSpec((BLK, HID // 2), lambda i, be: (i, 0)),
        scratch_shapes=[
            pltpu.VMEM((HID, FF), jnp.bfloat16),
            pltpu.VMEM((HID, FF), jnp.bfloat16),
            pltpu.VMEM((FF, HID), jnp.bfloat16),
        ],
    )
    return pl.pallas_call(
        _group_body,
        grid_spec=grid_spec,
        out_shape=jax.ShapeDtypeStruct((RS, HID // 2), jnp.int32),
        compiler_params=pltpu.CompilerParams(
            dimension_semantics=("arbitrary",)),
    )(be, xs, w_gate, w_up, w_down)


def _shared_body(x_ref, wg_ref, wu_ref, wd_ref, d_ref, o_ref):
    del d_ref
    x = x_ref[...].astype(jnp.bfloat16)
    g = jnp.dot(x, wg_ref[...].astype(jnp.bfloat16),
                preferred_element_type=jnp.float32)
    u = jnp.dot(x, wu_ref[...].astype(jnp.bfloat16),
                preferred_element_type=jnp.float32)
    h = (g * lax.logistic(g) * u).astype(jnp.bfloat16)
    o_ref[...] = jnp.dot(h, wd_ref[...].astype(jnp.bfloat16),
                         preferred_element_type=jnp.float32)


def _shared_part(x, sw_gate, sw_up, sw_down, dep, dep_spec, off, nb):
    rb = 256
    return pl.pallas_call(
        _shared_body,
        grid=(nb,),
        in_specs=[
            pl.BlockSpec((rb, HID), lambda i: (i + off, 0)),
            pl.BlockSpec((HID, SFF), lambda i: (0, 0)),
            pl.BlockSpec((HID, SFF), lambda i: (0, 0)),
            pl.BlockSpec((SFF, HID), lambda i: (0, 0)),
            dep_spec,
        ],
        out_specs=pl.BlockSpec((rb, HID), lambda i: (i, 0)),
        out_shape=jax.ShapeDtypeStruct((nb * rb, HID), jnp.float32),
        compiler_params=pltpu.CompilerParams(
            dimension_semantics=("arbitrary",)),
    )(x, sw_gate, sw_up, sw_down, dep)


def _combine_body(s_ref, g0_ref, g1_ref, w1_ref, w2_ref, o_ref):
    g0 = _unpack2(g0_ref[...]).astype(jnp.float32)
    g1 = _unpack2(g1_ref[...]).astype(jnp.float32)
    o_ref[...] = s_ref[...] + w1_ref[...] * g0 + w2_ref[...] * g1


def _combine(sh, garr, w1, w2):
    nb = 4
    rb = T // nb
    return pl.pallas_call(
        _combine_body,
        grid=(nb,),
        in_specs=[
            pl.BlockSpec((rb, HID), lambda i: (i, 0)),
            pl.BlockSpec((rb, HID // 2), lambda i: (i, 0)),
            pl.BlockSpec((rb, HID // 2), lambda i: (i + nb, 0)),
            pl.BlockSpec((rb, 1), lambda i: (i, 0)),
            pl.BlockSpec((rb, 1), lambda i: (i, 0)),
        ],
        out_specs=pl.BlockSpec((rb, HID), lambda i: (i, 0)),
        out_shape=jax.ShapeDtypeStruct((T, HID), jnp.float32),
        compiler_params=pltpu.CompilerParams(
            dimension_semantics=("arbitrary",)),
    )(sh, garr, garr, w1, w2)


def kernel(hidden_states, visual_token_mask, gate_w, e_score_bias,
           w_gate, w_up, w_down, sw_gate, sw_up, sw_down):
    del visual_token_mask
    x = hidden_states.reshape(T, HID)
    p1, p2, w1, w2, be, xb = _router(x, gate_w, e_score_bias.reshape(1, E))
    idx = jnp.concatenate([p1[:, 0], p2[:, 0]])
    xs = _sc_scatter(xb, idx)
    ys = _grouped(be[:, 0], xs, w_gate, w_up, w_down)
    garr = _sc_gather(ys, idx)
    sh = _shared_part(x, sw_gate, sw_up, sw_down, ys,
                      pl.BlockSpec((8, 128), lambda i: (0, 0)), 0, 8)
    out = _combine(sh, garr, w1, w2)
    return out.reshape(hidden_states.shape)


# speedup vs baseline: 1.0909x; 1.0661x over previous
import jax
import jax.numpy as jnp
from jax import lax
from jax.experimental import pallas as pl
from jax.experimental.pallas import tpu as pltpu
from jax.experimental.pallas import tpu_sc as plsc

T = 2048
HID = 1024
FF = 512
E = 8
K = 2
SFF = 1024
BLK = 256
NP = T * K
G = (NP + E * (BLK - 1) + BLK - 1) // BLK
RS = G * BLK

NW = 32
PW = NP // NW
CH = 64


def _pack2(xbf):
    half = xbf.shape[-1] // 2
    lo = pltpu.bitcast(xbf[:, :half], jnp.int16).astype(jnp.int32) & 0xFFFF
    hi = pltpu.bitcast(xbf[:, half:], jnp.int16).astype(jnp.int32) & 0xFFFF
    return lo | (hi << 16)


def _unpack2(xi):
    lo = pltpu.bitcast((xi & 0xFFFF).astype(jnp.int16), jnp.bfloat16)
    hi = pltpu.bitcast(((xi >> 16) & 0xFFFF).astype(jnp.int16), jnp.bfloat16)
    return jnp.concatenate([lo, hi], axis=-1)


def _router_body(x_ref, gw_ref, b_ref, p1_ref, p2_ref, w1_ref, w2_ref, be_ref,
                 xb_ref):
    x = x_ref[...]
    xb_ref[...] = _pack2(x.astype(jnp.bfloat16))
    logits = jnp.dot(x, gw_ref[...], preferred_element_type=jnp.float32)
    m = jnp.max(logits, axis=-1, keepdims=True)
    ex = jnp.exp(logits - m)
    scores = ex / jnp.sum(ex, axis=-1, keepdims=True)
    choice = scores + b_ref[...]
    lane = lax.broadcasted_iota(jnp.int32, (T, E), 1)
    v1 = jnp.max(choice, axis=-1, keepdims=True)
    e1 = jnp.min(jnp.where(choice == v1, lane, E), axis=-1, keepdims=True)
    ch2 = jnp.where(lane == e1, -jnp.inf, choice)
    v2 = jnp.max(ch2, axis=-1, keepdims=True)
    e2 = jnp.min(jnp.where(ch2 == v2, lane, E), axis=-1, keepdims=True)
    w1 = jnp.sum(jnp.where(lane == e1, scores, 0.0), axis=-1, keepdims=True)
    w2 = jnp.sum(jnp.where(lane == e2, scores, 0.0), axis=-1, keepdims=True)
    tot = w1 + w2
    w1_ref[...] = w1 / tot
    w2_ref[...] = w2 / tot
    ohc = jnp.concatenate(
        [(lane == e1).astype(jnp.bfloat16), (lane == e2).astype(jnp.bfloat16)],
        axis=-1)
    CK = 128
    NCK = T // CK
    rc = lax.broadcasted_iota(jnp.int32, (CK, CK), 0)
    cc = lax.broadcasted_iota(jnp.int32, (CK, CK), 1)
    tric = (cc < rc).astype(jnp.bfloat16)
    locs, tots = [], []
    for ci in range(NCK):
        blk = ohc[ci * CK:(ci + 1) * CK, :]
        loc = jnp.dot(tric, blk, preferred_element_type=jnp.float32)
        locs.append(loc)
        tots.append(loc[CK - 1:CK, :] + blk[CK - 1:CK, :].astype(jnp.float32))
    tot = jnp.concatenate(tots, axis=0)
    r16 = lax.broadcasted_iota(jnp.int32, (NCK, NCK), 0)
    c16 = lax.broadcasted_iota(jnp.int32, (NCK, NCK), 1)
    tri16 = (c16 < r16).astype(jnp.float32)
    off = jnp.dot(tri16, tot, preferred_element_type=jnp.float32)
    rank = jnp.concatenate(
        [locs[ci] + off[ci:ci + 1, :] for ci in range(NCK)], axis=0)
    c1 = rank[:, :E]
    c2 = rank[:, E:]
    allcnt = off[NCK - 1:NCK, :] + tot[NCK - 1:NCK, :]
    cnt1 = allcnt[:, :E]
    counts = cnt1 + allcnt[:, E:]
    pc = jnp.maximum(jnp.ceil(counts * (1.0 / BLK)) * BLK, float(BLK))
    r8 = lax.broadcasted_iota(jnp.int32, (E, E), 0)
    c8 = lax.broadcasted_iota(jnp.int32, (E, E), 1)
    tri8 = (r8 < c8).astype(jnp.float32)
    seg = jnp.dot(pc, tri8, preferred_element_type=jnp.float32)
    p1 = jnp.sum(jnp.where(lane == e1, seg + c1, 0.0), axis=-1, keepdims=True)
    p2 = jnp.sum(jnp.where(lane == e2, seg + cnt1 + c2, 0.0), axis=-1,
                 keepdims=True)
    p1_ref[...] = p1.astype(jnp.int32)
    p2_ref[...] = p2.astype(jnp.int32)
    gb = lax.broadcasted_iota(jnp.int32, (G + 1, E), 0) * BLK
    segi = seg.astype(jnp.int32)
    be = jnp.sum((gb >= segi).astype(jnp.int32), axis=-1, keepdims=True) - 1
    nb = (segi[0, E - 1] + pc.astype(jnp.int32)[0, E - 1]) // BLK
    row = lax.broadcasted_iota(jnp.int32, (G + 1, 1), 0)
    be_ref[...] = jnp.where(row == G, nb, be)


def _router(x, gate_w, bias2d):
    return pl.pallas_call(
        _router_body,
        out_shape=(
            jax.ShapeDtypeStruct((T, 1), jnp.int32),
            jax.ShapeDtypeStruct((T, 1), jnp.int32),
            jax.ShapeDtypeStruct((T, 1), jnp.float32),
            jax.ShapeDtypeStruct((T, 1), jnp.float32),
            jax.ShapeDtypeStruct((G + 1, 1), jnp.int32),
            jax.ShapeDtypeStruct((T, HID // 2), jnp.int32),
        ),
    )(x, gate_w, bias2d)


def _sc_scatter(x, idx):
    mesh = plsc.VectorSubcoreMesh(core_axis_name="c", subcore_axis_name="s")

    @pl.kernel(
        out_type=jax.ShapeDtypeStruct((RS, HID // 2), jnp.int32),
        mesh=mesh,
        scratch_types=[
            pltpu.VMEM((CH,), jnp.int32),
            pltpu.VMEM((CH, HID // 2), jnp.int32),
            pltpu.SemaphoreType.DMA,
        ],
    )
    def _disp(x_hbm, idx_hbm, xs_hbm, idx_v, rows_v, sem):
        cid = lax.axis_index("c")
        sid = lax.axis_index("s")
        wid = sid * 2 + cid
        base = wid * PW
        for chk in range(PW // CH):
            off = base + chk * CH
            pltpu.sync_copy(idx_hbm.at[pl.ds(off, CH)], idx_v)
            src = lax.rem(off, T)
            pltpu.sync_copy(x_hbm.at[pl.ds(src, CH)], rows_v)
            pltpu.async_copy(rows_v, xs_hbm.at[idx_v], sem).wait()

    return _disp(x, idx)


def _sc_gather(ys, idx):
    mesh = plsc.VectorSubcoreMesh(core_axis_name="c", subcore_axis_name="s")

    @pl.kernel(
        out_type=jax.ShapeDtypeStruct((NP, HID // 2), jnp.int32),
        mesh=mesh,
        scratch_types=[
            pltpu.VMEM((CH,), jnp.int32),
            pltpu.VMEM((CH, HID // 2), jnp.int32),
            pltpu.SemaphoreType.DMA,
        ],
    )
    def _gath(ys_hbm, idx_hbm, g_hbm, idx_v, rows_v, sem):
        cid = lax.axis_index("c")
        sid = lax.axis_index("s")
        wid = sid * 2 + cid
        base = wid * PW
        for chk in range(PW // CH):
            off = base + chk * CH
            pltpu.sync_copy(idx_hbm.at[pl.ds(off, CH)], idx_v)
            pltpu.async_copy(ys_hbm.at[idx_v], rows_v, sem).wait()
            pltpu.sync_copy(rows_v, g_hbm.at[pl.ds(off, CH)])

    return _gath(ys, idx)


def _group_body(be_ref, xs_ref, wg_ref, wu_ref, wd_ref, ys_ref,
                wg_stg, wu_stg, wd_stg, wgb_ref, wub_ref, wdb_ref, sem):
    i = pl.program_id(0)
    nb = be_ref[G]
    e = be_ref[i]
    prev_e = be_ref[jnp.maximum(i - 1, 0)]

    @pl.when(i == 0)
    def _prime():
        for slot in (0, 1):
            pltpu.make_async_copy(wg_ref.at[slot], wg_stg.at[slot],
                                  sem.at[slot]).start()
            pltpu.make_async_copy(wu_ref.at[slot], wu_stg.at[slot],
                                  sem.at[slot]).start()
            pltpu.make_async_copy(wd_ref.at[slot], wd_stg.at[slot],
                                  sem.at[slot]).start()

    @pl.when(((i == 0) | (e != prev_e)) & (i < nb))
    def _advance_expert():
        slot = e % 2
        pltpu.make_async_copy(wg_ref.at[e], wg_stg.at[slot],
                              sem.at[slot]).wait()
        pltpu.make_async_copy(wu_ref.at[e], wu_stg.at[slot],
                              sem.at[slot]).wait()
        pltpu.make_async_copy(wd_ref.at[e], wd_stg.at[slot],
                              sem.at[slot]).wait()
        wgb_ref[...] = wg_stg[slot].astype(jnp.bfloat16)
        wub_ref[...] = wu_stg[slot].astype(jnp.bfloat16)
        wdb_ref[...] = wd_stg[slot].astype(jnp.bfloat16)

        @pl.when(e + 2 < E)
        def _prefetch_next():
            pltpu.make_async_copy(wg_ref.at[e + 2], wg_stg.at[slot],
                                  sem.at[slot]).start()
            pltpu.make_async_copy(wu_ref.at[e + 2], wu_stg.at[slot],
                                  sem.at[slot]).start()
            pltpu.make_async_copy(wd_ref.at[e + 2], wd_stg.at[slot],
                                  sem.at[slot]).start()

    @pl.when(i < nb)
    def _compute():
        x = _unpack2(xs_ref[...])
        g = jnp.dot(x, wgb_ref[...], preferred_element_type=jnp.float32)
        u = jnp.dot(x, wub_ref[...], preferred_element_type=jnp.float32)
        h = (g * lax.logistic(g) * u).astype(jnp.bfloat16)
        y = jnp.dot(h, wdb_ref[...], preferred_element_type=jnp.float32)
        ys_ref[...] = _pack2(y.astype(jnp.bfloat16))


def _grouped(be, xs, w_gate, w_up, w_down):
    grid_spec = pltpu.PrefetchScalarGridSpec(
        num_scalar_prefetch=1,
        grid=(G,),
        in_specs=[
            pl.BlockSpec((BLK, HID // 2), lambda i, be: (i, 0)),
            pl.BlockSpec(memory_space=pltpu.MemorySpace.HBM),
            pl.BlockSpec(memory_space=pltpu.MemorySpace.HBM),
            pl.BlockSpec(memory_space=pltpu.MemorySpace.HBM),
        ],
        out_specs=pl.BlockSpec((BLK, HID // 2), lambda i, be: (i, 0)),
        scratch_shapes=[
            pltpu.VMEM((2, HID, FF), jnp.float32),
            pltpu.VMEM((2, HID, FF), jnp.float32),
            pltpu.VMEM((2, FF, HID), jnp.float32),
            pltpu.VMEM((HID, FF), jnp.bfloat16),
            pltpu.VMEM((HID, FF), jnp.bfloat16),
            pltpu.VMEM((FF, HID), jnp.bfloat16),
            pltpu.SemaphoreType.DMA((2,)),
        ],
    )
    return pl.pallas_call(
        _group_body,
        grid_spec=grid_spec,
        out_shape=jax.ShapeDtypeStruct((RS, HID // 2), jnp.int32),
        compiler_params=pltpu.CompilerParams(
            dimension_semantics=("arbitrary",)),
    )(be, xs, w_gate, w_up, w_down)


def _shared_body(x_ref, wg_ref, wu_ref, wd_ref, d_ref, o_ref):
    del d_ref
    x = x_ref[...].astype(jnp.bfloat16)
    g = jnp.dot(x, wg_ref[...].astype(jnp.bfloat16),
                preferred_element_type=jnp.float32)
    u = jnp.dot(x, wu_ref[...].astype(jnp.bfloat16),
                preferred_element_type=jnp.float32)
    h = (g * lax.logistic(g) * u).astype(jnp.bfloat16)
    o_ref[...] = jnp.dot(h, wd_ref[...].astype(jnp.bfloat16),
                         preferred_element_type=jnp.float32)


def _shared_part(x, sw_gate, sw_up, sw_down, dep, dep_spec, off, nb):
    rb = 256
    return pl.pallas_call(
        _shared_body,
        grid=(nb,),
        in_specs=[
            pl.BlockSpec((rb, HID), lambda i: (i + off, 0)),
            pl.BlockSpec((HID, SFF), lambda i: (0, 0)),
            pl.BlockSpec((HID, SFF), lambda i: (0, 0)),
            pl.BlockSpec((SFF, HID), lambda i: (0, 0)),
            dep_spec,
        ],
        out_specs=pl.BlockSpec((rb, HID), lambda i: (i, 0)),
        out_shape=jax.ShapeDtypeStruct((nb * rb, HID), jnp.float32),
        compiler_params=pltpu.CompilerParams(
            dimension_semantics=("arbitrary",)),
    )(x, sw_gate, sw_up, sw_down, dep)


def _combine_body(s_ref, g0_ref, g1_ref, w1_ref, w2_ref, o_ref):
    g0 = _unpack2(g0_ref[...]).astype(jnp.float32)
    g1 = _unpack2(g1_ref[...]).astype(jnp.float32)
    o_ref[...] = s_ref[...] + w1_ref[...] * g0 + w2_ref[...] * g1


def _combine(sh, garr, w1, w2):
    nb = 4
    rb = T // nb
    return pl.pallas_call(
        _combine_body,
        grid=(nb,),
        in_specs=[
            pl.BlockSpec((rb, HID), lambda i: (i, 0)),
            pl.BlockSpec((rb, HID // 2), lambda i: (i, 0)),
            pl.BlockSpec((rb, HID // 2), lambda i: (i + nb, 0)),
            pl.BlockSpec((rb, 1), lambda i: (i, 0)),
            pl.BlockSpec((rb, 1), lambda i: (i, 0)),
        ],
        out_specs=pl.BlockSpec((rb, HID), lambda i: (i, 0)),
        out_shape=jax.ShapeDtypeStruct((T, HID), jnp.float32),
        compiler_params=pltpu.CompilerParams(
            dimension_semantics=("arbitrary",)),
    )(sh, garr, garr, w1, w2)


def kernel(hidden_states, visual_token_mask, gate_w, e_score_bias,
           w_gate, w_up, w_down, sw_gate, sw_up, sw_down):
    del visual_token_mask
    x = hidden_states.reshape(T, HID)
    p1, p2, w1, w2, be, xb = _router(x, gate_w, e_score_bias.reshape(1, E))
    idx = jnp.concatenate([p1[:, 0], p2[:, 0]])
    xs = _sc_scatter(xb, idx)
    ys = _grouped(be[:, 0], xs, w_gate, w_up, w_down)
    garr = _sc_gather(ys, idx)
    sh = _shared_part(x, sw_gate, sw_up, sw_down, ys,
                      pl.BlockSpec((8, 128), lambda i: (0, 0)), 0, 8)
    out = _combine(sh, garr, w1, w2)
    return out.reshape(hidden_states.shape)
